# baseline (speedup 1.0000x reference)
import functools

import jax
import jax.numpy as jnp
from jax import lax
from jax.experimental import pallas as pl
from jax.experimental.pallas import tpu as pltpu
from jax.experimental.pallas import tpu_sc as plsc

VOCAB = 1000
NUM_CORES = 2
NUM_SUBCORES = 16
NUM_WORKERS = NUM_CORES * NUM_SUBCORES


def _make_gather(n_idx: int, chunk: int):
    per_w = n_idx // NUM_WORKERS
    n_chunks = per_w // chunk
    assert per_w % chunk == 0 and chunk % 8 == 0

    mesh = plsc.VectorSubcoreMesh(core_axis_name="c", subcore_axis_name="s")

    @functools.partial(
        pl.kernel,
        mesh=mesh,
        compiler_params=pltpu.CompilerParams(use_tc_tiling_on_sc=False),
        out_type=jax.ShapeDtypeStruct((n_idx, VOCAB), jnp.float32),
        scratch_types=[
            pltpu.VMEM((per_w,), jnp.int32),
            pltpu.VMEM((2, chunk, VOCAB), jnp.float32),
            pltpu.SemaphoreType.DMA,
        ],
    )
    def gather_kernel(tok_hbm, table_hbm, out_hbm, idx_v, rows_v, sem):
        wid = lax.axis_index("s") * NUM_CORES + lax.axis_index("c")
        base = wid * per_w
        pltpu.sync_copy(tok_hbm.at[pl.ds(base, per_w)], idx_v)

        def body(c, _):
            idx_slice = idx_v.at[pl.ds(c * chunk, chunk)]
            pltpu.async_copy(table_hbm.at[idx_slice], rows_v.at[0], sem).wait()
            pltpu.sync_copy(rows_v.at[0], out_hbm.at[pl.ds(base + c * chunk, chunk)])
            return ()

        lax.fori_loop(0, n_chunks, body, ())

    return gather_kernel


def kernel(tokens, lookup_table):
    b, s = tokens.shape
    n_idx = b * s
    flat = tokens.reshape(n_idx).astype(jnp.int32)
    out = _make_gather(n_idx, chunk=40)(flat, lookup_table)
    return out.reshape(b, s, VOCAB)


# speedup vs baseline: 1.1677x; 1.1677x over previous
import functools

import jax
import jax.numpy as jnp
from jax import lax
from jax.experimental import pallas as pl
from jax.experimental.pallas import tpu as pltpu
from jax.experimental.pallas import tpu_sc as plsc

V = 1000
S = 50
NC = 2
NS = 16
NW = NC * NS
Q = 4
CB = 32

mesh = plsc.VectorSubcoreMesh(core_axis_name="c", subcore_axis_name="s")


@functools.partial(
    pl.kernel,
    mesh=mesh,
    compiler_params=pltpu.CompilerParams(
        use_tc_tiling_on_sc=False, needs_layout_passes=False
    ),
    out_type=jax.ShapeDtypeStruct((S, V // 8, 8, 8, 128), jnp.float32),
    scratch_types=[
        pltpu.VMEM((S, CB), jnp.int32),
        pltpu.VMEM((CB, V), jnp.float32),
        pltpu.VMEM((CB, V), jnp.float32),
        pltpu.VMEM((V // 8, 8, CB), jnp.float32),
        pltpu.SemaphoreType.DMA,
        pltpu.SemaphoreType.DMA,
        pltpu.SemaphoreType.DMA,
    ],
)
def _bigram(tokT, table, out, idxbuf, rows0, rows1, tbuf, semg0, semg1, semw):
    wid = lax.axis_index("s") * NC + lax.axis_index("c")
    bt = wid // Q
    q = wid % Q
    col0 = bt * 128 + q * CB

    pltpu.sync_copy(tokT.at[:, pl.ds(col0, CB)], idxbuf)

    def gather_start(k, rows, semg):
        pltpu.async_copy(table.at[idxbuf.at[k]], rows, semg)

    def gather_wait(k, rows, semg):
        pltpu.make_async_copy(table.at[idxbuf.at[k]], rows, semg).wait()

    def out_ref(s):
        return out.at[s, :, bt, :, pl.ds(q * CB, CB)]

    iota = lax.iota(jnp.int32, 16)

    def transpose(rows):
        def tbody(vt, _):
            for vi in range(8):
                colv = jnp.zeros((16,), jnp.int32) + (vt * 8 + vi)
                for half in range(2):
                    vals = plsc.load_gather(rows, [iota + half * 16, colv])
                    tbuf[vt, vi, pl.ds(half * 16, 16)] = vals
            return 0

        lax.fori_loop(0, V // 8, tbody, 0)

    gather_start(0, rows0, semg0)

    def body(j, _):
        for b, (cur, semc, nxt, semn) in enumerate(
            ((rows0, semg0, rows1, semg1), (rows1, semg1, rows0, semg0))
        ):
            k = j * 2 + b
            gather_wait(k, cur, semc)

            @pl.when(k < S - 1)
            def _():
                gather_start(k + 1, nxt, semn)

            @pl.when(k > 0)
            def _():
                pltpu.make_async_copy(tbuf, out_ref(jnp.maximum(k - 1, 0)), semw).wait()

            transpose(cur)
            pltpu.async_copy(tbuf, out_ref(k), semw)
        return 0

    lax.fori_loop(0, S // 2, body, 0)
    pltpu.make_async_copy(tbuf, out_ref(S - 1), semw).wait()


def kernel(tokens, lookup_table):
    b, s = tokens.shape
    tokT = tokens.T.astype(jnp.int32)
    x5 = _bigram(tokT, lookup_table)
    return jnp.transpose(x5, (2, 4, 0, 1, 3)).reshape(b, s, V)


# speedup vs baseline: 3.7680x; 3.2269x over previous
import functools

import jax
import jax.numpy as jnp
from jax import lax
from jax.experimental import pallas as pl
from jax.experimental.pallas import tpu as pltpu
from jax.experimental.pallas import tpu_sc as plsc

V = 1000
S = 50
NC = 2
NS = 16
NW = NC * NS
Q = 4
CB = 32

mesh = plsc.VectorSubcoreMesh(core_axis_name="c", subcore_axis_name="s")


@functools.partial(
    pl.kernel,
    mesh=mesh,
    compiler_params=pltpu.CompilerParams(
        use_tc_tiling_on_sc=False, needs_layout_passes=False
    ),
    out_type=jax.ShapeDtypeStruct((S, V // 8, 8, 8, 128), jnp.float32),
    scratch_types=[
        pltpu.VMEM((S, CB), jnp.int32),
        pltpu.VMEM((CB, V), jnp.float32),
        pltpu.VMEM((CB, V), jnp.float32),
        pltpu.VMEM((V // 8, 8, CB), jnp.float32),
        pltpu.SemaphoreType.DMA,
        pltpu.SemaphoreType.DMA,
        pltpu.SemaphoreType.DMA,
    ],
)
def _bigram(tokT, table, out, idxbuf, rows0, rows1, tbuf, semg0, semg1, semw):
    wid = lax.axis_index("s") * NC + lax.axis_index("c")
    bt = wid // Q
    q = wid % Q
    col0 = bt * 128 + q * CB

    pltpu.sync_copy(tokT.at[:, pl.ds(col0, CB)], idxbuf)

    def gather_start(k, rows, semg):
        pltpu.async_copy(table.at[idxbuf.at[k]], rows, semg)

    def gather_wait(k, rows, semg):
        pltpu.make_async_copy(table.at[idxbuf.at[k]], rows, semg).wait()

    def out_ref(s):
        return out.at[s, :, bt, :, pl.ds(q * CB, CB)]

    iota = lax.iota(jnp.int32, 16)

    def transpose(rows):
        @plsc.parallel_loop(0, V // 8, unroll=2)
        def tbody(vt):
            for vi in range(8):
                colv = jnp.zeros((16,), jnp.int32) + (vt * 8 + vi)
                for half in range(2):
                    vals = plsc.load_gather(rows, [iota + half * 16, colv])
                    tbuf[vt, vi, pl.ds(half * 16, 16)] = vals

    gather_start(0, rows0, semg0)

    def body(j, _):
        for b, (cur, semc, nxt, semn) in enumerate(
            ((rows0, semg0, rows1, semg1), (rows1, semg1, rows0, semg0))
        ):
            k = j * 2 + b
            gather_wait(k, cur, semc)

            @pl.when(k < S - 1)
            def _():
                gather_start(k + 1, nxt, semn)

            @pl.when(k > 0)
            def _():
                pltpu.make_async_copy(tbuf, out_ref(jnp.maximum(k - 1, 0)), semw).wait()

            transpose(cur)
            pltpu.async_copy(tbuf, out_ref(k), semw)
        return 0

    lax.fori_loop(0, S // 2, body, 0)
    pltpu.make_async_copy(tbuf, out_ref(S - 1), semw).wait()


def kernel(tokens, lookup_table):
    b, s = tokens.shape
    tokT = tokens.T.astype(jnp.int32)
    x5 = _bigram(tokT, lookup_table)
    return jnp.transpose(x5, (2, 4, 0, 1, 3)).reshape(b, s, V)
